# baseline (speedup 1.0000x reference)
import functools

import jax
import jax.numpy as jnp
from jax import lax
from jax.experimental import pallas as pl
from jax.experimental.pallas import tpu as pltpu
from jax.experimental.pallas import tpu_sc as plsc

_C = 128


@functools.lru_cache(maxsize=None)
def _build(n_rows, d):
    info = plsc.get_sparse_core_info()
    nw = info.num_cores * info.num_subcores
    n_full = n_rows // _C
    tail = n_rows - n_full * _C
    n_chunks = n_full + (1 if tail else 0)
    iters = -(-n_chunks // nw)

    mesh = plsc.VectorSubcoreMesh(core_axis_name="c", subcore_axis_name="s")

    @functools.partial(
        pl.kernel,
        mesh=mesh,
        out_type=(
            jax.ShapeDtypeStruct((n_rows, d), jnp.float32),
            jax.ShapeDtypeStruct((n_rows, d), jnp.float32),
        ),
        scratch_types=[
            pltpu.VMEM((_C,), jnp.int32),
            pltpu.VMEM((_C,), jnp.int32),
            pltpu.VMEM((_C, d), jnp.float32),
            pltpu.VMEM((_C, d), jnp.float32),
            pltpu.SemaphoreType.DMA,
            pltpu.SemaphoreType.DMA,
        ],
    )
    def gather_kernel(data_hbm, idxp_hbm, idxn_hbm, outp_hbm, outn_hbm,
                      idxp_v, idxn_v, rowsp_v, rowsn_v, semp, semn):
        wid = lax.axis_index("s") * info.num_cores + lax.axis_index("c")

        def step(i, carry):
            c = wid + i * nw

            @pl.when(c < n_full)
            def _full():
                r0 = c * _C
                pltpu.sync_copy(idxp_hbm.at[pl.ds(r0, _C)], idxp_v)
                pltpu.sync_copy(idxn_hbm.at[pl.ds(r0, _C)], idxn_v)
                cp = pltpu.async_copy(data_hbm.at[idxp_v], rowsp_v, semp)
                cn = pltpu.async_copy(data_hbm.at[idxn_v], rowsn_v, semn)
                cp.wait()
                cn.wait()
                pltpu.sync_copy(rowsp_v, outp_hbm.at[pl.ds(r0, _C)])
                pltpu.sync_copy(rowsn_v, outn_hbm.at[pl.ds(r0, _C)])

            if tail:
                @pl.when(c == n_full)
                def _tail():
                    r0 = n_full * _C
                    pltpu.sync_copy(idxp_hbm.at[pl.ds(r0, tail)],
                                    idxp_v.at[pl.ds(0, tail)])
                    pltpu.sync_copy(idxn_hbm.at[pl.ds(r0, tail)],
                                    idxn_v.at[pl.ds(0, tail)])
                    cp = pltpu.async_copy(data_hbm.at[idxp_v.at[pl.ds(0, tail)]],
                                          rowsp_v.at[pl.ds(0, tail)], semp)
                    cn = pltpu.async_copy(data_hbm.at[idxn_v.at[pl.ds(0, tail)]],
                                          rowsn_v.at[pl.ds(0, tail)], semn)
                    cp.wait()
                    cn.wait()
                    pltpu.sync_copy(rowsp_v.at[pl.ds(0, tail)],
                                    outp_hbm.at[pl.ds(r0, tail)])
                    pltpu.sync_copy(rowsn_v.at[pl.ds(0, tail)],
                                    outn_hbm.at[pl.ds(r0, tail)])

            return carry

        lax.fori_loop(0, iters, step, 0)

    return gather_kernel


def kernel(data, edge_index_pos, edge_index_neg):
    n, d = data.shape
    e = edge_index_pos.shape[1]
    idxp = edge_index_pos.astype(jnp.int32).T.reshape(-1)
    idxn = edge_index_neg.astype(jnp.int32).T.reshape(-1)
    outp, outn = _build(2 * e, d)(data, idxp, idxn)
    return outp.reshape(e, 2 * d), outn.reshape(e, 2 * d)


# speedup vs baseline: 1.0336x; 1.0336x over previous
import functools

import jax
import jax.numpy as jnp
from jax import lax
from jax.experimental import pallas as pl
from jax.experimental.pallas import tpu as pltpu
from jax.experimental.pallas import tpu_sc as plsc

_C = 128
_NBUF = 3


@functools.lru_cache(maxsize=None)
def _build(n_rows, d):
    info = plsc.get_sparse_core_info()
    nc = info.num_cores
    nw = nc * info.num_subcores
    n_full = n_rows // _C
    tail = n_rows - n_full * _C
    n_chunks = n_full + (1 if tail else 0)
    cpw = -(-n_chunks // nw)
    n_pad_rows = nw * cpw * _C
    rounds = -(-cpw // _NBUF) + 1

    mesh = plsc.VectorSubcoreMesh(core_axis_name="c", subcore_axis_name="s")

    @functools.partial(
        pl.kernel,
        mesh=mesh,
        out_type=(
            jax.ShapeDtypeStruct((n_rows, d), jnp.float32),
            jax.ShapeDtypeStruct((n_rows, d), jnp.float32),
        ),
        scratch_types=(
            [pltpu.VMEM((cpw * _C,), jnp.int32) for _ in range(2)]
            + [pltpu.VMEM((_C, d), jnp.float32) for _ in range(2 * _NBUF)]
            + [pltpu.SemaphoreType.DMA for _ in range(4 * _NBUF)]
        ),
    )
    def gather_kernel(data_hbm, idxp_hbm, idxn_hbm, outp_hbm, outn_hbm, *sc):
        idx_v = (sc[0], sc[1])
        rows = [[sc[2 + s * _NBUF + b] for b in range(_NBUF)] for s in range(2)]
        o = 2 + 2 * _NBUF
        gsem = [[sc[o + s * _NBUF + b] for b in range(_NBUF)] for s in range(2)]
        o += 2 * _NBUF
        wsem = [[sc[o + s * _NBUF + b] for b in range(_NBUF)] for s in range(2)]
        outs = (outp_hbm, outn_hbm)

        wid = lax.axis_index("s") * nc + lax.axis_index("c")
        base_chunk = wid * cpw
        pltpu.sync_copy(idxp_hbm.at[pl.ds(base_chunk * _C, cpw * _C)], idx_v[0])
        pltpu.sync_copy(idxn_hbm.at[pl.ds(base_chunk * _C, cpw * _C)], idx_v[1])

        def write_desc(s, b, cg, wait):
            @pl.when(cg < n_full)
            def _full():
                cp = pltpu.make_async_copy(
                    rows[s][b], outs[s].at[pl.ds(cg * _C, _C)], wsem[s][b])
                cp.wait() if wait else cp.start()
            if tail:
                @pl.when(cg == n_full)
                def _part():
                    cp = pltpu.make_async_copy(
                        rows[s][b].at[pl.ds(0, tail)],
                        outs[s].at[pl.ds(n_full * _C, tail)], wsem[s][b])
                    cp.wait() if wait else cp.start()

        def gather_desc(s, b, c_rel, wait):
            cp = pltpu.make_async_copy(
                data_hbm.at[idx_v[s].at[pl.ds(c_rel * _C, _C)]],
                rows[s][b], gsem[s][b])
            cp.wait() if wait else cp.start()

        def round_(j, carry):
            for b in range(_NBUF):
                c_new = j * _NBUF + b
                c_mid = c_new - _NBUF

                @pl.when(jnp.logical_and(c_mid >= 0, c_mid < cpw))
                def _drain_and_write():
                    for s in (0, 1):
                        gather_desc(s, b, c_mid, wait=True)
                    for s in (0, 1):
                        write_desc(s, b, base_chunk + c_mid, wait=False)

                @pl.when(c_new < cpw)
                def _fire():
                    @pl.when(c_mid >= 0)
                    def _wait_prev_write():
                        for s in (0, 1):
                            write_desc(s, b, base_chunk + c_mid, wait=True)
                    for s in (0, 1):
                        gather_desc(s, b, c_new, wait=False)

            return carry

        lax.fori_loop(0, rounds, round_, 0)

        for b in range(_NBUF):
            last_c = ((cpw - 1 - b) // _NBUF) * _NBUF + b
            if last_c >= 0:
                for s in (0, 1):
                    write_desc(s, b, base_chunk + last_c, wait=True)

    return gather_kernel, n_pad_rows, cpw


def kernel(data, edge_index_pos, edge_index_neg):
    n, d = data.shape
    e = edge_index_pos.shape[1]
    fn, n_pad, _ = _build(2 * e, d)
    pad = n_pad - 2 * e

    def prep(ei):
        flat = ei.astype(jnp.int32).T.reshape(-1)
        return jnp.concatenate([flat, jnp.zeros((pad,), jnp.int32)])

    outp, outn = fn(data, prep(edge_index_pos), prep(edge_index_neg))
    return outp.reshape(e, 2 * d), outn.reshape(e, 2 * d)


# speedup vs baseline: 2.9661x; 2.8698x over previous
import functools

import jax
import jax.numpy as jnp
from jax import lax
from jax.experimental import pallas as pl
from jax.experimental.pallas import tpu as pltpu
from jax.experimental.pallas import tpu_sc as plsc

_EPC = 64
_NBUF = 3


@functools.lru_cache(maxsize=None)
def _build(e, d):
    info = plsc.get_sparse_core_info()
    nc = info.num_cores
    nw = nc * info.num_subcores
    n_full = e // _EPC
    tail = e - n_full * _EPC
    n_chunks = n_full + (1 if tail else 0)
    cpw = -(-n_chunks // nw)
    epw = cpw * _EPC
    pad = nw * epw - e
    rounds = -(-cpw // _NBUF) + 1

    mesh = plsc.VectorSubcoreMesh(core_axis_name="c", subcore_axis_name="s")

    @functools.partial(
        pl.kernel,
        mesh=mesh,
        out_type=(
            jax.ShapeDtypeStruct((e, 2 * d), jnp.float32),
            jax.ShapeDtypeStruct((e, 2 * d), jnp.float32),
        ),
        scratch_types=(
            [pltpu.VMEM((epw,), jnp.int32) for _ in range(4)]
            + [pltpu.VMEM((_EPC, 2 * d), jnp.float32) for _ in range(2 * _NBUF)]
            + [pltpu.SemaphoreType.DMA for _ in range(4 * _NBUF)]
        ),
    )
    def gather_kernel(data_hbm, eip_hbm, ein_hbm, outp_hbm, outn_hbm, *sc):
        ed = [[sc[2 * s + h] for h in range(2)] for s in range(2)]
        rows = [[sc[4 + s * _NBUF + b] for b in range(_NBUF)] for s in range(2)]
        o = 4 + 2 * _NBUF
        gsem = [[sc[o + s * _NBUF + b] for b in range(_NBUF)] for s in range(2)]
        o += 2 * _NBUF
        wsem = [[sc[o + s * _NBUF + b] for b in range(_NBUF)] for s in range(2)]
        outs = (outp_hbm, outn_hbm)
        eis = (eip_hbm, ein_hbm)

        wid = lax.axis_index("s") * nc + lax.axis_index("c")
        base_chunk = wid * cpw
        base_e = wid * epw

        for s in range(2):
            for h in range(2):
                pltpu.async_copy(eis[s].at[pl.ds(h * e + base_e, epw)],
                                 ed[s][h], gsem[s][h])
        for s in range(2):
            for h in range(2):
                pltpu.make_async_copy(eis[s].at[pl.ds(h * e + base_e, epw)],
                                      ed[s][h], gsem[s][h]).wait()

        def write_desc(s, b, cg, wait):
            @pl.when(cg < n_full)
            def _full():
                cp = pltpu.make_async_copy(
                    rows[s][b], outs[s].at[pl.ds(cg * _EPC, _EPC)], wsem[s][b])
                cp.wait() if wait else cp.start()
            if tail:
                @pl.when(cg == n_full)
                def _part():
                    cp = pltpu.make_async_copy(
                        rows[s][b].at[pl.ds(0, tail)],
                        outs[s].at[pl.ds(n_full * _EPC, tail)], wsem[s][b])
                    cp.wait() if wait else cp.start()

        def gather_desc(s, b, c_rel, wait):
            for h in range(2):
                cp = pltpu.make_async_copy(
                    data_hbm.at[ed[s][h].at[pl.ds(c_rel * _EPC, _EPC)]],
                    rows[s][b].at[:, pl.ds(h * d, d)], gsem[s][b])
                cp.wait() if wait else cp.start()

        def round_(j, carry):
            for b in range(_NBUF):
                c_new = j * _NBUF + b
                c_mid = c_new - _NBUF

                @pl.when(jnp.logical_and(c_mid >= 0, c_mid < cpw))
                def _drain_and_write():
                    for s in (0, 1):
                        gather_desc(s, b, c_mid, wait=True)
                    for s in (0, 1):
                        write_desc(s, b, base_chunk + c_mid, wait=False)

                @pl.when(c_new < cpw)
                def _fire():
                    @pl.when(c_mid >= 0)
                    def _wait_prev_write():
                        for s in (0, 1):
                            write_desc(s, b, base_chunk + c_mid, wait=True)
                    for s in (0, 1):
                        gather_desc(s, b, c_new, wait=False)

            return carry

        lax.fori_loop(0, rounds, round_, 0)

        for b in range(_NBUF):
            last_c = ((cpw - 1 - b) // _NBUF) * _NBUF + b
            if last_c >= 0:
                for s in (0, 1):
                    write_desc(s, b, base_chunk + last_c, wait=True)

    return gather_kernel, pad


def kernel(data, edge_index_pos, edge_index_neg):
    n, d = data.shape
    e = edge_index_pos.shape[1]
    fn, pad = _build(e, d)
    zpad = jnp.zeros((pad,), jnp.int32)

    def prep(ei):
        return jnp.concatenate([ei.astype(jnp.int32).reshape(-1), zpad])

    return fn(data, prep(edge_index_pos), prep(edge_index_neg))
